# Initial kernel scaffold; baseline (speedup 1.0000x reference)
import jax
import jax.numpy as jnp
from jax.experimental import pallas as pl


def kernel(x, table):
    raise NotImplementedError("write your pallas kernel here")


# speedup vs baseline: 1.4914x; 1.4914x over previous
import functools

import jax
import jax.numpy as jnp
from jax import lax
from jax.experimental import pallas as pl
from jax.experimental.pallas import tpu as pltpu
from jax.experimental.pallas import tpu_sc as plsc

_D = 32
_NC = 2
_NS = 16
_NW = _NC * _NS
_CH = 128
_NBUF = 4


def _make_gather(B):
    assert B % (_NW * _CH) == 0
    cpw = B // (_NW * _CH)

    @functools.partial(
        pl.kernel,
        out_type=jax.ShapeDtypeStruct((B, _D), jnp.float32),
        mesh=plsc.VectorSubcoreMesh(core_axis_name="c", subcore_axis_name="s"),
        compiler_params=pltpu.CompilerParams(use_tc_tiling_on_sc=False),
        scratch_types=(
            [pltpu.VMEM((cpw, _CH), jnp.int32),
             pltpu.VMEM((_NBUF, _CH, _D), jnp.float32)]
            + [pltpu.SemaphoreType.DMA] * (2 * _NBUF)
        ),
    )
    def gather_kernel(x_hbm, tab_hbm, out_hbm, idx_v, rows_v, *sems):
        gsems = sems[:_NBUF]
        wsems = sems[_NBUF:]
        wid = lax.axis_index("s") * _NC + lax.axis_index("c")
        rbase = wid * cpw
        obase = wid * cpw * _CH

        pltpu.sync_copy(x_hbm.at[pl.ds(rbase, cpw)], idx_v)

        for b in range(_NBUF):
            pltpu.async_copy(tab_hbm.at[idx_v.at[b]], rows_v.at[b], gsems[b])

        def step(g, carry):
            for b in range(_NBUF):
                j = g * _NBUF + b
                pltpu.make_async_copy(
                    tab_hbm.at[idx_v.at[j]], rows_v.at[b], gsems[b]).wait()
                dst = out_hbm.at[pl.ds(obase + j * _CH, _CH)]
                pltpu.async_copy(rows_v.at[b], dst, wsems[b])
                pltpu.make_async_copy(rows_v.at[b], dst, wsems[b]).wait()
                nj = j + _NBUF

                @pl.when(nj < cpw)
                def _():
                    pltpu.async_copy(
                        tab_hbm.at[idx_v.at[nj]], rows_v.at[b], gsems[b])
            return carry

        lax.fori_loop(0, cpw // _NBUF, step, 0)

    return gather_kernel


def kernel(x, table):
    batch, hist = x.shape
    B = batch * hist
    xf = x.reshape(-1).astype(jnp.int32).reshape(B // _CH, _CH)
    out = _make_gather(B)(xf, table)
    return out.reshape(batch, hist, _D)
